# baseline (speedup 1.0000x reference)
import functools

import jax
import jax.numpy as jnp
from jax import lax
from jax.experimental import pallas as pl
from jax.experimental.pallas import tpu as pltpu
from jax.experimental.pallas import tpu_sc as plsc

B = 4
S = 2048
D = 2048
NR = 16
NVT = 100
TOPK = 2
BS = 512
NSS = S // BS

NC = 2
NSC = 16
NW = NC * NSC
WPB = NW // B


def _route_body(x_ref, mt_ref, w_ref, pk_ref):
    b = pl.program_id(0)
    x = x_ref[0]
    lane = lax.broadcasted_iota(jnp.int32, (1, B), 1)
    sel = (lane == b).astype(jnp.float32)
    m = jnp.sum(mt_ref[...] * sel, axis=1, keepdims=True)
    ssum = jnp.sum(x * m, axis=0, keepdims=True)
    cnt = jnp.maximum(jnp.sum(m), 1.0)
    sent = ssum / cnt
    logits = lax.dot_general(sent, w_ref[...], (((1,), (1,)), ((), ())),
                             preferred_element_type=jnp.float32,
                             precision=lax.Precision.HIGHEST)
    mx = jnp.max(logits, axis=1, keepdims=True)
    e = jnp.exp(logits - mx)
    p = e / jnp.sum(e, axis=1, keepdims=True)
    iota = lax.broadcasted_iota(jnp.int32, (1, NR), 1)
    m1 = jnp.max(p, axis=1, keepdims=True)
    i1 = jnp.min(jnp.where(p == m1, iota, NR), axis=1, keepdims=True)
    p2 = jnp.where(iota == i1, -1.0, p)
    m2 = jnp.max(p2, axis=1, keepdims=True)
    i2 = jnp.min(jnp.where(p2 == m2, iota, NR), axis=1, keepdims=True)
    pk = jnp.where(iota == 0, m1, 0.0)
    pk = jnp.where(iota == 1, m2, pk)
    pk = jnp.where(iota == 2, (i1 * NVT).astype(jnp.float32), pk)
    pk = jnp.where(iota == 3, (i2 * NVT).astype(jnp.float32), pk)
    pk_ref[...] = pk.reshape(1, 1, NR)


def _route(inputs_embeds, mask_t, W_router):
    return pl.pallas_call(
        _route_body,
        grid=(B,),
        in_specs=[
            pl.BlockSpec((1, S, D), lambda b: (b, 0, 0)),
            pl.BlockSpec((S, B), lambda b: (0, 0)),
            pl.BlockSpec((NR, D), lambda b: (0, 0)),
        ],
        out_specs=pl.BlockSpec((1, 1, NR), lambda b: (b, 0, 0)),
        out_shape=jax.ShapeDtypeStruct((B, 1, NR), jnp.float32),
    )(inputs_embeds, mask_t, W_router)


def _combine_body(pk_hbm, emb_hbm, out_hbm,
                  pk_v, idx0_v, idx1_v, r0_v, r1_v, o_v,
                  s0, s1, s2, s3, s4):
    cid = lax.axis_index("c")
    sid = lax.axis_index("s")
    wid = sid * NC + cid
    b = wid // WPB
    lc = wid % WPB
    start_r = jnp.where(lc < 5, lc * 16, jnp.where(lc < 7, 40 + 8 * lc, 96))

    pltpu.async_copy(pk_hbm, pk_v, s0).wait()

    pkrow = pk_v[b, 0, :]
    w0 = jnp.full((16,), pkrow[0], jnp.float32)
    w1 = jnp.full((16,), pkrow[1], jnp.float32)
    o0 = pkrow[2].astype(jnp.int32)
    o1 = pkrow[3].astype(jnp.int32)

    r = jnp.minimum(start_r + lax.iota(jnp.int32, 16), NVT - 1)
    idx0_v[...] = o0 + r
    idx1_v[...] = o1 + r

    def gather(lo, n, sa, sb):
        c0 = pltpu.async_copy(emb_hbm.at[idx0_v.at[pl.ds(lo, n)]],
                              r0_v.at[pl.ds(lo, n)], sa)
        c1 = pltpu.async_copy(emb_hbm.at[idx1_v.at[pl.ds(lo, n)]],
                              r1_v.at[pl.ds(lo, n)], sb)
        return c0, c1

    def combine(lo, n):
        for row in range(lo, lo + n):
            @plsc.parallel_loop(0, D // 16, unroll=16)
            def body(c, row=row):
                sl = pl.ds(c * 16, 16)
                o_v[row, sl] = r0_v[row, sl] * w0 + r1_v[row, sl] * w1

    def store(lo, n, sem):
        return pltpu.async_copy(o_v.at[pl.ds(lo, n)],
                                out_hbm.at[b, pl.ds(start_r + lo, n)], sem)

    @pl.when(lc < 5)
    def _():
        a0, a1 = gather(0, 8, s0, s1)
        b0, b1 = gather(8, 8, s2, s3)
        a0.wait()
        a1.wait()
        combine(0, 8)
        st0 = store(0, 8, s4)
        b0.wait()
        b1.wait()
        combine(8, 8)
        st1 = store(8, 8, s0)
        st0.wait()
        st1.wait()

    @pl.when((lc >= 5) & (lc < 7))
    def _():
        a0, a1 = gather(0, 8, s0, s1)
        a0.wait()
        a1.wait()
        combine(0, 8)
        st = store(0, 8, s4)
        st.wait()

    @pl.when(lc == 7)
    def _():
        a0, a1 = gather(0, 8, s0, s1)
        a0.wait()
        a1.wait()
        combine(0, 4)
        st = store(0, 4, s4)
        st.wait()


@functools.lru_cache(maxsize=1)
def _combine():
    return pl.kernel(
        _combine_body,
        mesh=plsc.VectorSubcoreMesh(core_axis_name="c", subcore_axis_name="s"),
        out_type=jax.ShapeDtypeStruct((B, NVT, D), jnp.float32),
        scratch_types=[
            pltpu.VMEM((B, 1, NR), jnp.float32),
            pltpu.VMEM((16,), jnp.int32),
            pltpu.VMEM((16,), jnp.int32),
            pltpu.VMEM((16, D), jnp.float32),
            pltpu.VMEM((16, D), jnp.float32),
            pltpu.VMEM((16, D), jnp.float32),
            pltpu.SemaphoreType.DMA,
            pltpu.SemaphoreType.DMA,
            pltpu.SemaphoreType.DMA,
            pltpu.SemaphoreType.DMA,
            pltpu.SemaphoreType.DMA,
        ],
    )


def kernel(indices, input_ids, inputs_embeds, attention_mask, embedding, W_router):
    mask_t = attention_mask.astype(jnp.float32).T
    pk = _route(inputs_embeds, mask_t, W_router)
    return _combine()(pk, embedding)


# speedup vs baseline: 1.0308x; 1.0308x over previous
import functools

import jax
import jax.numpy as jnp
from jax import lax
from jax.experimental import pallas as pl
from jax.experimental.pallas import tpu as pltpu
from jax.experimental.pallas import tpu_sc as plsc

B = 4
S = 2048
D = 2048
NR = 16
NVT = 100
TOPK = 2
BS = 512
NSS = S // BS

NC = 2
NSC = 16
NW = NC * NSC
WPB = NW // B


def _route_body(x_ref, mt_ref, w_ref, pk_ref):
    b = pl.program_id(0)
    x = x_ref[0]
    lane = lax.broadcasted_iota(jnp.int32, (1, B), 1)
    sel = (lane == b).astype(jnp.float32)
    m = jnp.sum(mt_ref[...] * sel, axis=1, keepdims=True)
    ssum = jnp.sum(x * m, axis=0, keepdims=True)
    cnt = jnp.maximum(jnp.sum(m), 1.0)
    sent = ssum / cnt
    logits = lax.dot_general(sent, w_ref[...], (((1,), (1,)), ((), ())),
                             preferred_element_type=jnp.float32,
                             precision=lax.Precision.HIGHEST)
    mx = jnp.max(logits, axis=1, keepdims=True)
    e = jnp.exp(logits - mx)
    p = e / jnp.sum(e, axis=1, keepdims=True)
    iota = lax.broadcasted_iota(jnp.int32, (1, NR), 1)
    m1 = jnp.max(p, axis=1, keepdims=True)
    i1 = jnp.min(jnp.where(p == m1, iota, NR), axis=1, keepdims=True)
    p2 = jnp.where(iota == i1, -1.0, p)
    m2 = jnp.max(p2, axis=1, keepdims=True)
    i2 = jnp.min(jnp.where(p2 == m2, iota, NR), axis=1, keepdims=True)
    pk = jnp.where(iota == 0, m1, 0.0)
    pk = jnp.where(iota == 1, m2, pk)
    pk = jnp.where(iota == 2, (i1 * NVT).astype(jnp.float32), pk)
    pk = jnp.where(iota == 3, (i2 * NVT).astype(jnp.float32), pk)
    pk_ref[...] = pk.reshape(1, 1, NR)


def _route(inputs_embeds, mask_t, W_router):
    return pl.pallas_call(
        _route_body,
        grid=(B,),
        in_specs=[
            pl.BlockSpec((1, S, D), lambda b: (b, 0, 0)),
            pl.BlockSpec((S, B), lambda b: (0, 0)),
            pl.BlockSpec((NR, D), lambda b: (0, 0)),
        ],
        out_specs=pl.BlockSpec((1, 1, NR), lambda b: (b, 0, 0)),
        out_shape=jax.ShapeDtypeStruct((B, 1, NR), jnp.float32),
    )(inputs_embeds, mask_t, W_router)


def _combine_body(pk_hbm, emb_hbm, out_hbm,
                  pk_v, idx0_v, idx1_v, r0_v, r1_v, o_v,
                  s0, s1, s2, s3, s4):
    cid = lax.axis_index("c")
    sid = lax.axis_index("s")
    wid = sid * NC + cid
    b = wid // WPB
    lc = wid % WPB
    start_r = jnp.where(lc < 5, lc * 16, jnp.where(lc < 7, 40 + 8 * lc, 96))

    pltpu.async_copy(pk_hbm, pk_v, s0).wait()

    pkrow = pk_v[b, 0, :]
    w0 = jnp.full((16,), pkrow[0], jnp.float32)
    w1 = jnp.full((16,), pkrow[1], jnp.float32)
    o0 = pkrow[2].astype(jnp.int32)
    o1 = pkrow[3].astype(jnp.int32)

    r = jnp.minimum(start_r + lax.iota(jnp.int32, 16), NVT - 1)
    idx0_v[...] = o0 + r
    idx1_v[...] = o1 + r

    def gather(lo, n, sa, sb):
        c0 = pltpu.async_copy(emb_hbm.at[idx0_v.at[pl.ds(lo, n)]],
                              r0_v.at[pl.ds(lo, n)], sa)
        c1 = pltpu.async_copy(emb_hbm.at[idx1_v.at[pl.ds(lo, n)]],
                              r1_v.at[pl.ds(lo, n)], sb)
        return c0, c1

    def combine(lo, n):
        @plsc.parallel_loop(0, n * (D // 16), unroll=8)
        def body(c):
            row = lo + c // (D // 16)
            sl = pl.ds((c % (D // 16)) * 16, 16)
            o_v[row, sl] = r0_v[row, sl] * w0 + r1_v[row, sl] * w1

    def store(lo, n, sem):
        return pltpu.async_copy(o_v.at[pl.ds(lo, n)],
                                out_hbm.at[b, pl.ds(start_r + lo, n)], sem)

    @pl.when(lc < 5)
    def _():
        a0, a1 = gather(0, 8, s0, s1)
        b0, b1 = gather(8, 8, s2, s3)
        a0.wait()
        a1.wait()
        combine(0, 8)
        st0 = store(0, 8, s4)
        b0.wait()
        b1.wait()
        combine(8, 8)
        st1 = store(8, 8, s0)
        st0.wait()
        st1.wait()

    @pl.when((lc >= 5) & (lc < 7))
    def _():
        a0, a1 = gather(0, 8, s0, s1)
        a0.wait()
        a1.wait()
        combine(0, 8)
        st = store(0, 8, s4)
        st.wait()

    @pl.when(lc == 7)
    def _():
        a0, a1 = gather(0, 8, s0, s1)
        a0.wait()
        a1.wait()
        combine(0, 4)
        st = store(0, 4, s4)
        st.wait()


@functools.lru_cache(maxsize=1)
def _combine():
    return pl.kernel(
        _combine_body,
        mesh=plsc.VectorSubcoreMesh(core_axis_name="c", subcore_axis_name="s"),
        out_type=jax.ShapeDtypeStruct((B, NVT, D), jnp.float32),
        scratch_types=[
            pltpu.VMEM((B, 1, NR), jnp.float32),
            pltpu.VMEM((16,), jnp.int32),
            pltpu.VMEM((16,), jnp.int32),
            pltpu.VMEM((16, D), jnp.float32),
            pltpu.VMEM((16, D), jnp.float32),
            pltpu.VMEM((16, D), jnp.float32),
            pltpu.SemaphoreType.DMA,
            pltpu.SemaphoreType.DMA,
            pltpu.SemaphoreType.DMA,
            pltpu.SemaphoreType.DMA,
            pltpu.SemaphoreType.DMA,
        ],
    )


def kernel(indices, input_ids, inputs_embeds, attention_mask, embedding, W_router):
    mask_t = attention_mask.astype(jnp.float32).T
    pk = _route(inputs_embeds, mask_t, W_router)
    return _combine()(pk, embedding)


# speedup vs baseline: 1.0356x; 1.0047x over previous
import functools

import jax
import jax.numpy as jnp
from jax import lax
from jax.experimental import pallas as pl
from jax.experimental.pallas import tpu as pltpu
from jax.experimental.pallas import tpu_sc as plsc

B = 4
S = 2048
D = 2048
NR = 16
NVT = 100
TOPK = 2
BS = 512
NSS = S // BS

NC = 2
NSC = 16
NW = NC * NSC
WPB = NW // B


def _route_body(x_ref, mt_ref, w_ref, pk_ref):
    b = pl.program_id(0)
    x = x_ref[0]
    lane = lax.broadcasted_iota(jnp.int32, (1, B), 1)
    sel = (lane == b).astype(jnp.float32)
    m = jnp.sum(mt_ref[...] * sel, axis=1, keepdims=True)
    ssum = jnp.sum(x * m, axis=0, keepdims=True)
    cnt = jnp.maximum(jnp.sum(m), 1.0)
    sent = ssum / cnt
    logits = lax.dot_general(sent, w_ref[...], (((1,), (1,)), ((), ())),
                             preferred_element_type=jnp.float32,
                             precision=lax.Precision.HIGHEST)
    mx = jnp.max(logits, axis=1, keepdims=True)
    e = jnp.exp(logits - mx)
    p = e / jnp.sum(e, axis=1, keepdims=True)
    iota = lax.broadcasted_iota(jnp.int32, (1, NR), 1)
    m1 = jnp.max(p, axis=1, keepdims=True)
    i1 = jnp.min(jnp.where(p == m1, iota, NR), axis=1, keepdims=True)
    p2 = jnp.where(iota == i1, -1.0, p)
    m2 = jnp.max(p2, axis=1, keepdims=True)
    i2 = jnp.min(jnp.where(p2 == m2, iota, NR), axis=1, keepdims=True)
    pk = jnp.where(iota == 0, m1, 0.0)
    pk = jnp.where(iota == 1, m2, pk)
    pk = jnp.where(iota == 2, (i1 * NVT).astype(jnp.float32), pk)
    pk = jnp.where(iota == 3, (i2 * NVT).astype(jnp.float32), pk)
    pk_ref[...] = pk.reshape(1, 1, NR)


def _route(inputs_embeds, mask_t, W_router):
    return pl.pallas_call(
        _route_body,
        grid=(B,),
        in_specs=[
            pl.BlockSpec((1, S, D), lambda b: (b, 0, 0)),
            pl.BlockSpec((S, B), lambda b: (0, 0)),
            pl.BlockSpec((NR, D), lambda b: (0, 0)),
        ],
        out_specs=pl.BlockSpec((1, 1, NR), lambda b: (b, 0, 0)),
        out_shape=jax.ShapeDtypeStruct((B, 1, NR), jnp.float32),
    )(inputs_embeds, mask_t, W_router)


def _combine_body(pk_hbm, emb_hbm, out_hbm,
                  pk_v, idx0_v, idx1_v, r0_v, r1_v, o_v,
                  s0, s1, s2, s3, s4):
    cid = lax.axis_index("c")
    sid = lax.axis_index("s")
    wid = sid * NC + cid
    b = wid // WPB
    lc = wid % WPB
    start_r = jnp.where(lc < 5, lc * 16, jnp.where(lc < 7, 40 + 8 * lc, 96))

    pltpu.async_copy(pk_hbm, pk_v, s0).wait()

    pkrow = pk_v[b, 0, :]
    w0 = jnp.full((16,), pkrow[0], jnp.float32)
    w1 = jnp.full((16,), pkrow[1], jnp.float32)
    o0 = pkrow[2].astype(jnp.int32)
    o1 = pkrow[3].astype(jnp.int32)

    r = jnp.minimum(start_r + lax.iota(jnp.int32, 16), NVT - 1)
    idx0_v[...] = o0 + r
    idx1_v[...] = o1 + r

    def gather(lo, n, sa, sb):
        c0 = pltpu.async_copy(emb_hbm.at[idx0_v.at[pl.ds(lo, n)]],
                              r0_v.at[pl.ds(lo, n)], sa)
        c1 = pltpu.async_copy(emb_hbm.at[idx1_v.at[pl.ds(lo, n)]],
                              r1_v.at[pl.ds(lo, n)], sb)
        return c0, c1

    def combine(lo, n):
        @plsc.parallel_loop(0, n * (D // 16), unroll=4)
        def body(c):
            row = lo + c // (D // 16)
            sl = pl.ds((c % (D // 16)) * 16, 16)
            o_v[row, sl] = r0_v[row, sl] * w0 + r1_v[row, sl] * w1

    def store(lo, n, sem):
        return pltpu.async_copy(o_v.at[pl.ds(lo, n)],
                                out_hbm.at[b, pl.ds(start_r + lo, n)], sem)

    @pl.when(lc < 5)
    def _():
        a0, a1 = gather(0, 8, s0, s1)
        b0, b1 = gather(8, 8, s2, s3)
        a0.wait()
        a1.wait()
        combine(0, 8)
        st0 = store(0, 8, s4)
        b0.wait()
        b1.wait()
        combine(8, 8)
        st1 = store(8, 8, s0)
        st0.wait()
        st1.wait()

    @pl.when((lc >= 5) & (lc < 7))
    def _():
        a0, a1 = gather(0, 8, s0, s1)
        a0.wait()
        a1.wait()
        combine(0, 8)
        st = store(0, 8, s4)
        st.wait()

    @pl.when(lc == 7)
    def _():
        a0, a1 = gather(0, 8, s0, s1)
        a0.wait()
        a1.wait()
        combine(0, 4)
        st = store(0, 4, s4)
        st.wait()


@functools.lru_cache(maxsize=1)
def _combine():
    return pl.kernel(
        _combine_body,
        mesh=plsc.VectorSubcoreMesh(core_axis_name="c", subcore_axis_name="s"),
        out_type=jax.ShapeDtypeStruct((B, NVT, D), jnp.float32),
        scratch_types=[
            pltpu.VMEM((B, 1, NR), jnp.float32),
            pltpu.VMEM((16,), jnp.int32),
            pltpu.VMEM((16,), jnp.int32),
            pltpu.VMEM((16, D), jnp.float32),
            pltpu.VMEM((16, D), jnp.float32),
            pltpu.VMEM((16, D), jnp.float32),
            pltpu.SemaphoreType.DMA,
            pltpu.SemaphoreType.DMA,
            pltpu.SemaphoreType.DMA,
            pltpu.SemaphoreType.DMA,
            pltpu.SemaphoreType.DMA,
        ],
    )


def kernel(indices, input_ids, inputs_embeds, attention_mask, embedding, W_router):
    mask_t = attention_mask.astype(jnp.float32).T
    pk = _route(inputs_embeds, mask_t, W_router)
    return _combine()(pk, embedding)
